# Initial kernel scaffold; baseline (speedup 1.0000x reference)
import jax
import jax.numpy as jnp
from jax.experimental import pallas as pl


def kernel(boxes, scores, iou_threshold):
    raise NotImplementedError("write your pallas kernel here")


# speedup vs baseline: 59.5341x; 59.5341x over previous
import functools

import jax
import jax.numpy as jnp
from jax import lax
from jax.experimental import pallas as pl
from jax.experimental.pallas import tpu as pltpu
from jax.experimental.pallas import tpu_sc as plsc

_N = 5000
_B = 256
_NP = 5120
_NB = _NP // _B
_MAX_OUT = 1000
_OUT_PAD = 1024

_SC_CORES = 2
_SC_SUBCORES = 16
_SC_WORKERS = _SC_CORES * _SC_SUBCORES
_SC_SLOTS = _OUT_PAD // _SC_WORKERS


def _iou_tile(x1c, y1c, x2c, y2c, ac, x1r, y1r, x2r, y2r, ar):
    xx1 = jnp.maximum(x1c, x1r)
    yy1 = jnp.maximum(y1c, y1r)
    xx2 = jnp.minimum(x2c, x2r)
    yy2 = jnp.minimum(y2c, y2r)
    inter = jnp.clip(xx2 - xx1, 0.0) * jnp.clip(yy2 - yy1, 0.0)
    return inter / (ac + ar - inter + 1e-9)


def _supp_body(thr_ref, cold_ref, rowd_ref, rowb_ref, c_ref, supp_ref,
               supbb_ref, cnt_ref):
    p = pl.program_id(0)
    thr = thr_ref[0]

    @pl.when(p == 0)
    def _init():
        gidx = (lax.broadcasted_iota(jnp.int32, (_NB, _B), 0) * _B
                + lax.broadcasted_iota(jnp.int32, (_NB, _B), 1))
        supp_ref[:, :] = jnp.where(gidx < _N, 0.0, 1.0)
        cnt_ref[0] = 0

    x1c = cold_ref[:, 0:1]
    y1c = cold_ref[:, 1:2]
    x2c = cold_ref[:, 2:3]
    y2c = cold_ref[:, 3:4]
    ac = cold_ref[:, 4:5]

    iou_bb = _iou_tile(x1c, y1c, x2c, y2c, ac,
                       rowb_ref[0:1, :], rowb_ref[1:2, :], rowb_ref[2:3, :],
                       rowb_ref[3:4, :], rowb_ref[4:5, :])
    tri = (lax.broadcasted_iota(jnp.int32, (_B, _B), 0)
           < lax.broadcasted_iota(jnp.int32, (_B, _B), 1))
    supbb_ref[:, :] = jnp.where((iou_bb > thr) & tri, 1.0, 0.0)

    kb0 = 1.0 - supp_ref[pl.ds(p, 1), :]
    lane = lax.broadcasted_iota(jnp.int32, (1, _B), 1)

    def _resolve(i, kb):
        row = supbb_ref[pl.ds(i, 1), :]
        kbi = jnp.sum(jnp.where(lane == i, kb, 0.0))
        return kb * (1.0 - row * kbi)

    kb = lax.fori_loop(0, _B, _resolve, kb0)

    cnt0 = cnt_ref[0]
    tri_le = jnp.where(lax.broadcasted_iota(jnp.int32, (_B, _B), 0)
                       <= lax.broadcasted_iota(jnp.int32, (_B, _B), 1),
                       1.0, 0.0)
    csum = jnp.dot(kb, tri_le, preferred_element_type=jnp.float32)
    c_ref[0, :, :] = csum.astype(jnp.int32) + cnt0
    cnt_ref[0] = cnt0 + jnp.sum(kb).astype(jnp.int32)

    for cb in range(1, _NB):
        @pl.when(cb > p)
        def _prop():
            s = cb * _B
            sf = jnp.where(
                _iou_tile(x1c, y1c, x2c, y2c, ac,
                          rowd_ref[0:1, s:s + _B], rowd_ref[1:2, s:s + _B],
                          rowd_ref[2:3, s:s + _B], rowd_ref[3:4, s:s + _B],
                          rowd_ref[4:5, s:s + _B]) > thr,
                1.0, 0.0)
            hits = jnp.dot(kb, sf, preferred_element_type=jnp.float32)
            old = supp_ref[pl.ds(cb, 1), :]
            supp_ref[pl.ds(cb, 1), :] = jnp.maximum(old, jnp.where(hits > 0.0, 1.0, 0.0))


def _run_suppression(coldata, rowdata, thr):
    return pl.pallas_call(
        _supp_body,
        grid=(_NB,),
        in_specs=[
            pl.BlockSpec(memory_space=pltpu.SMEM),
            pl.BlockSpec((_B, 5), lambda p: (p, 0)),
            pl.BlockSpec((5, _NP), lambda p: (0, 0)),
            pl.BlockSpec((5, _B), lambda p: (0, p)),
        ],
        out_specs=pl.BlockSpec((1, 1, _B), lambda p: (p, 0, 0)),
        out_shape=jax.ShapeDtypeStruct((_NB, 1, _B), jnp.int32),
        scratch_shapes=[
            pltpu.VMEM((_NB, _B), jnp.float32),
            pltpu.VMEM((_B, _B), jnp.float32),
            pltpu.SMEM((1,), jnp.int32),
        ],
    )(thr, coldata, rowdata, rowdata)


def _compact_body(c_hbm, orig_hbm, out_hbm, c_v, o_v, res_v):
    wid = lax.axis_index("s") * _SC_CORES + lax.axis_index("c")
    pltpu.sync_copy(c_hbm, c_v)
    pltpu.sync_copy(orig_hbm, o_v)
    last_idx = jnp.full((16,), _NP - 1, jnp.int32)
    c_last = plsc.load_gather(c_v, [last_idx])
    for g in range(_SC_SLOTS // 16):
        tgt = wid * _SC_SLOTS + g * 16 + lax.iota(jnp.int32, 16) + 1

        def _bstep(_, carry):
            lo, hi = carry
            live = lo < hi
            mid = jnp.minimum((lo + hi) // 2, _NP - 1)
            v = plsc.load_gather(c_v, [mid])
            pred = v >= tgt
            lo2 = jnp.where(live & jnp.logical_not(pred), mid + 1, lo)
            hi2 = jnp.where(live & pred, mid, hi)
            return lo2, hi2

        lo0 = jnp.zeros((16,), jnp.int32)
        hi0 = jnp.full((16,), _NP, jnp.int32)
        pos, _ = lax.fori_loop(0, 13, _bstep, (lo0, hi0))
        valid = c_last >= tgt
        safe = jnp.minimum(pos, _NP - 1)
        ov = plsc.load_gather(o_v, [safe])
        res_v[pl.ds(g * 16, 16)] = jnp.where(valid, ov, -1)
    pltpu.sync_copy(res_v, out_hbm.at[pl.ds(wid * _SC_SLOTS, _SC_SLOTS)])


@functools.cache
def _compact_call():
    return pl.kernel(
        _compact_body,
        out_type=jax.ShapeDtypeStruct((_OUT_PAD,), jnp.int32),
        mesh=plsc.VectorSubcoreMesh(core_axis_name="c", subcore_axis_name="s"),
        compiler_params=pltpu.CompilerParams(needs_layout_passes=False),
        scratch_types=[
            pltpu.VMEM((_NP,), jnp.int32),
            pltpu.VMEM((_NP,), jnp.int32),
            pltpu.VMEM((_SC_SLOTS,), jnp.int32),
        ],
    )


def kernel(boxes, scores, iou_threshold):
    order = jnp.argsort(-scores)
    bs = boxes[order]
    pad = _NP - _N
    x1 = jnp.pad(bs[:, 0], (0, pad))
    y1 = jnp.pad(bs[:, 1], (0, pad))
    x2 = jnp.pad(bs[:, 2], (0, pad))
    y2 = jnp.pad(bs[:, 3], (0, pad))
    area = (x2 - x1) * (y2 - y1)
    rowdata = jnp.stack([x1, y1, x2, y2, area])
    coldata = rowdata.T
    thr = jnp.reshape(iou_threshold.astype(jnp.float32), (1,))

    c = _run_suppression(coldata, rowdata, thr).reshape(_NP)
    orig = jnp.pad(order.astype(jnp.int32), (0, pad))
    out = _compact_call()(c, orig)
    return out[:_MAX_OUT]


# speedup vs baseline: 359.3822x; 6.0366x over previous
import functools

import jax
import jax.numpy as jnp
from jax import lax
from jax.experimental import pallas as pl
from jax.experimental.pallas import tpu as pltpu
from jax.experimental.pallas import tpu_sc as plsc

_N = 5000
_B = 256
_NP = 5120
_NB = _NP // _B
_MAX_OUT = 1000
_OUT_PAD = 1024

_SC_CORES = 2
_SC_SUBCORES = 16
_SC_WORKERS = _SC_CORES * _SC_SUBCORES
_SC_SLOTS = _OUT_PAD // _SC_WORKERS


def _iou_tile(x1c, y1c, x2c, y2c, ac, x1r, y1r, x2r, y2r, ar):
    xx1 = jnp.maximum(x1c, x1r)
    yy1 = jnp.maximum(y1c, y1r)
    xx2 = jnp.minimum(x2c, x2r)
    yy2 = jnp.minimum(y2c, y2r)
    inter = jnp.clip(xx2 - xx1, 0.0) * jnp.clip(yy2 - yy1, 0.0)
    return inter / (ac + ar - inter + 1e-9)


def _supp_body(thr_ref, cold_ref, rowd_ref, rowb_ref, c_ref, supp_ref,
               cnt_ref):
    p = pl.program_id(0)
    thr = thr_ref[0]

    @pl.when(p == 0)
    def _init():
        gidx = (lax.broadcasted_iota(jnp.int32, (_NB, _B), 0) * _B
                + lax.broadcasted_iota(jnp.int32, (_NB, _B), 1))
        supp_ref[:, :] = jnp.where(gidx < _N, 0.0, 1.0)
        cnt_ref[0] = 0

    x1c = cold_ref[:, 0:1]
    y1c = cold_ref[:, 1:2]
    x2c = cold_ref[:, 2:3]
    y2c = cold_ref[:, 3:4]
    ac = cold_ref[:, 4:5]

    iou_bb = _iou_tile(x1c, y1c, x2c, y2c, ac,
                       rowb_ref[0:1, :], rowb_ref[1:2, :], rowb_ref[2:3, :],
                       rowb_ref[3:4, :], rowb_ref[4:5, :])
    tri = (lax.broadcasted_iota(jnp.int32, (_B, _B), 0)
           < lax.broadcasted_iota(jnp.int32, (_B, _B), 1))
    supbb = jnp.where((iou_bb > thr) & tri, 1.0, 0.0)

    kb0 = 1.0 - supp_ref[pl.ds(p, 1), :]

    def _rcond(carry):
        _, changed = carry
        return changed

    def _rbody(carry):
        kb, _ = carry
        hit = jnp.dot(kb, supbb, preferred_element_type=jnp.float32)
        kb_new = jnp.where(hit > 0.0, 0.0, kb0)
        return kb_new, jnp.any(kb_new != kb)

    kb, _ = lax.while_loop(_rcond, _rbody, (kb0, True))

    cnt0 = cnt_ref[0]
    tri_le = jnp.where(lax.broadcasted_iota(jnp.int32, (_B, _B), 0)
                       <= lax.broadcasted_iota(jnp.int32, (_B, _B), 1),
                       1.0, 0.0)
    csum = jnp.dot(kb, tri_le, preferred_element_type=jnp.float32)
    c_ref[0, :, :] = csum.astype(jnp.int32) + cnt0
    cnt_ref[0] = cnt0 + jnp.sum(kb).astype(jnp.int32)

    for cb in range(1, _NB):
        @pl.when(cb > p)
        def _prop():
            s = cb * _B
            sf = jnp.where(
                _iou_tile(x1c, y1c, x2c, y2c, ac,
                          rowd_ref[0:1, s:s + _B], rowd_ref[1:2, s:s + _B],
                          rowd_ref[2:3, s:s + _B], rowd_ref[3:4, s:s + _B],
                          rowd_ref[4:5, s:s + _B]) > thr,
                1.0, 0.0)
            hits = jnp.dot(kb, sf, preferred_element_type=jnp.float32)
            old = supp_ref[pl.ds(cb, 1), :]
            supp_ref[pl.ds(cb, 1), :] = jnp.maximum(old, jnp.where(hits > 0.0, 1.0, 0.0))


def _run_suppression(coldata, rowdata, thr):
    return pl.pallas_call(
        _supp_body,
        grid=(_NB,),
        in_specs=[
            pl.BlockSpec(memory_space=pltpu.SMEM),
            pl.BlockSpec((_B, 5), lambda p: (p, 0)),
            pl.BlockSpec((5, _NP), lambda p: (0, 0)),
            pl.BlockSpec((5, _B), lambda p: (0, p)),
        ],
        out_specs=pl.BlockSpec((1, 1, _B), lambda p: (p, 0, 0)),
        out_shape=jax.ShapeDtypeStruct((_NB, 1, _B), jnp.int32),
        scratch_shapes=[
            pltpu.VMEM((_NB, _B), jnp.float32),
            pltpu.SMEM((1,), jnp.int32),
        ],
    )(thr, coldata, rowdata, rowdata)


def _compact_body(c_hbm, orig_hbm, out_hbm, c_v, o_v, res_v):
    wid = lax.axis_index("s") * _SC_CORES + lax.axis_index("c")
    pltpu.sync_copy(c_hbm, c_v)
    pltpu.sync_copy(orig_hbm, o_v)
    last_idx = jnp.full((16,), _NP - 1, jnp.int32)
    c_last = plsc.load_gather(c_v, [last_idx])
    for g in range(_SC_SLOTS // 16):
        tgt = wid * _SC_SLOTS + g * 16 + lax.iota(jnp.int32, 16) + 1

        def _bstep(_, carry):
            lo, hi = carry
            live = lo < hi
            mid = jnp.minimum((lo + hi) // 2, _NP - 1)
            v = plsc.load_gather(c_v, [mid])
            pred = v >= tgt
            lo2 = jnp.where(live & jnp.logical_not(pred), mid + 1, lo)
            hi2 = jnp.where(live & pred, mid, hi)
            return lo2, hi2

        lo0 = jnp.zeros((16,), jnp.int32)
        hi0 = jnp.full((16,), _NP, jnp.int32)
        pos, _ = lax.fori_loop(0, 13, _bstep, (lo0, hi0))
        valid = c_last >= tgt
        safe = jnp.minimum(pos, _NP - 1)
        ov = plsc.load_gather(o_v, [safe])
        res_v[pl.ds(g * 16, 16)] = jnp.where(valid, ov, -1)
    pltpu.sync_copy(res_v, out_hbm.at[pl.ds(wid * _SC_SLOTS, _SC_SLOTS)])


@functools.cache
def _compact_call():
    return pl.kernel(
        _compact_body,
        out_type=jax.ShapeDtypeStruct((_OUT_PAD,), jnp.int32),
        mesh=plsc.VectorSubcoreMesh(core_axis_name="c", subcore_axis_name="s"),
        compiler_params=pltpu.CompilerParams(needs_layout_passes=False),
        scratch_types=[
            pltpu.VMEM((_NP,), jnp.int32),
            pltpu.VMEM((_NP,), jnp.int32),
            pltpu.VMEM((_SC_SLOTS,), jnp.int32),
        ],
    )


def kernel(boxes, scores, iou_threshold):
    order = jnp.argsort(-scores)
    bs = boxes[order]
    pad = _NP - _N
    x1 = jnp.pad(bs[:, 0], (0, pad))
    y1 = jnp.pad(bs[:, 1], (0, pad))
    x2 = jnp.pad(bs[:, 2], (0, pad))
    y2 = jnp.pad(bs[:, 3], (0, pad))
    area = (x2 - x1) * (y2 - y1)
    rowdata = jnp.stack([x1, y1, x2, y2, area])
    coldata = rowdata.T
    thr = jnp.reshape(iou_threshold.astype(jnp.float32), (1,))

    c = _run_suppression(coldata, rowdata, thr).reshape(_NP)
    orig = jnp.pad(order.astype(jnp.int32), (0, pad))
    out = _compact_call()(c, orig)
    return out[:_MAX_OUT]


# speedup vs baseline: 502.2080x; 1.3974x over previous
import functools

import jax
import jax.numpy as jnp
from jax import lax
from jax.experimental import pallas as pl
from jax.experimental.pallas import tpu as pltpu
from jax.experimental.pallas import tpu_sc as plsc

_N = 5000
_B = 256
_NP = 5120
_NB = _NP // _B
_MAX_OUT = 1000
_OUT_PAD = 1024

_SC_CORES = 2
_SC_SUBCORES = 16
_SC_WORKERS = _SC_CORES * _SC_SUBCORES
_SC_SLOTS = _OUT_PAD // _SC_WORKERS


def _iou_tile(x1c, y1c, x2c, y2c, ac, x1r, y1r, x2r, y2r, ar):
    xx1 = jnp.maximum(x1c, x1r)
    yy1 = jnp.maximum(y1c, y1r)
    xx2 = jnp.minimum(x2c, x2r)
    yy2 = jnp.minimum(y2c, y2r)
    inter = jnp.clip(xx2 - xx1, 0.0) * jnp.clip(yy2 - yy1, 0.0)
    return inter / (ac + ar - inter + 1e-9)


def _supp_body(thr_ref, cold_ref, rowd_ref, rowb_ref, c_ref, supp_ref,
               cnt_ref):
    p = pl.program_id(0)
    thr = thr_ref[0]

    @pl.when(p == 0)
    def _init():
        gidx = (lax.broadcasted_iota(jnp.int32, (_NB, _B), 0) * _B
                + lax.broadcasted_iota(jnp.int32, (_NB, _B), 1))
        supp_ref[:, :] = jnp.where(gidx < _N, 0.0, 1.0)
        cnt_ref[0] = 0

    cnt0 = cnt_ref[0]

    @pl.when(cnt0 >= _MAX_OUT)
    def _skip():
        c_ref[0, :, :] = jnp.full((1, _B), cnt0, jnp.int32)

    @pl.when(cnt0 < _MAX_OUT)
    def _work():
        x1c = cold_ref[:, 0:1]
        y1c = cold_ref[:, 1:2]
        x2c = cold_ref[:, 2:3]
        y2c = cold_ref[:, 3:4]
        ac = cold_ref[:, 4:5]

        iou_bb = _iou_tile(x1c, y1c, x2c, y2c, ac,
                           rowb_ref[0:1, :], rowb_ref[1:2, :], rowb_ref[2:3, :],
                           rowb_ref[3:4, :], rowb_ref[4:5, :])
        tri = (lax.broadcasted_iota(jnp.int32, (_B, _B), 0)
               < lax.broadcasted_iota(jnp.int32, (_B, _B), 1))
        supbb = jnp.where((iou_bb > thr) & tri, 1.0, 0.0)

        kb0 = 1.0 - supp_ref[pl.ds(p, 1), :]

        def _rcond(carry):
            _, changed = carry
            return changed

        def _rbody(carry):
            kb, _ = carry
            hit = jnp.dot(kb, supbb, preferred_element_type=jnp.float32)
            kb_new = jnp.where(hit > 0.0, 0.0, kb0)
            return kb_new, jnp.any(kb_new != kb)

        kb, _ = lax.while_loop(_rcond, _rbody, (kb0, True))

        tri_le = jnp.where(lax.broadcasted_iota(jnp.int32, (_B, _B), 0)
                           <= lax.broadcasted_iota(jnp.int32, (_B, _B), 1),
                           1.0, 0.0)
        csum = jnp.dot(kb, tri_le, preferred_element_type=jnp.float32)
        c_ref[0, :, :] = csum.astype(jnp.int32) + cnt0
        cnt_ref[0] = cnt0 + jnp.sum(kb).astype(jnp.int32)

        for cb in range(1, _NB):
            @pl.when(cb > p)
            def _prop():
                s = cb * _B
                sf = jnp.where(
                    _iou_tile(x1c, y1c, x2c, y2c, ac,
                              rowd_ref[0:1, s:s + _B], rowd_ref[1:2, s:s + _B],
                              rowd_ref[2:3, s:s + _B], rowd_ref[3:4, s:s + _B],
                              rowd_ref[4:5, s:s + _B]) > thr,
                    1.0, 0.0)
                hits = jnp.dot(kb, sf, preferred_element_type=jnp.float32)
                old = supp_ref[pl.ds(cb, 1), :]
                supp_ref[pl.ds(cb, 1), :] = jnp.maximum(
                    old, jnp.where(hits > 0.0, 1.0, 0.0))


def _run_suppression(coldata, rowdata, thr):
    return pl.pallas_call(
        _supp_body,
        grid=(_NB,),
        in_specs=[
            pl.BlockSpec(memory_space=pltpu.SMEM),
            pl.BlockSpec((_B, 5), lambda p: (p, 0)),
            pl.BlockSpec((5, _NP), lambda p: (0, 0)),
            pl.BlockSpec((5, _B), lambda p: (0, p)),
        ],
        out_specs=pl.BlockSpec((1, 1, _B), lambda p: (p, 0, 0)),
        out_shape=jax.ShapeDtypeStruct((_NB, 1, _B), jnp.int32),
        scratch_shapes=[
            pltpu.VMEM((_NB, _B), jnp.float32),
            pltpu.SMEM((1,), jnp.int32),
        ],
    )(thr, coldata, rowdata, rowdata)


def _compact_body(c_hbm, orig_hbm, out_hbm, c_v, o_v, res_v):
    wid = lax.axis_index("s") * _SC_CORES + lax.axis_index("c")
    pltpu.sync_copy(c_hbm, c_v)
    pltpu.sync_copy(orig_hbm, o_v)
    last_idx = jnp.full((16,), _NP - 1, jnp.int32)
    c_last = plsc.load_gather(c_v, [last_idx])
    for g in range(_SC_SLOTS // 16):
        tgt = wid * _SC_SLOTS + g * 16 + lax.iota(jnp.int32, 16) + 1

        def _bstep(_, carry):
            lo, hi = carry
            live = lo < hi
            mid = jnp.minimum((lo + hi) // 2, _NP - 1)
            v = plsc.load_gather(c_v, [mid])
            pred = v >= tgt
            lo2 = jnp.where(live & jnp.logical_not(pred), mid + 1, lo)
            hi2 = jnp.where(live & pred, mid, hi)
            return lo2, hi2

        lo0 = jnp.zeros((16,), jnp.int32)
        hi0 = jnp.full((16,), _NP, jnp.int32)
        pos, _ = lax.fori_loop(0, 13, _bstep, (lo0, hi0))
        valid = c_last >= tgt
        safe = jnp.minimum(pos, _NP - 1)
        ov = plsc.load_gather(o_v, [safe])
        res_v[pl.ds(g * 16, 16)] = jnp.where(valid, ov, -1)
    pltpu.sync_copy(res_v, out_hbm.at[pl.ds(wid * _SC_SLOTS, _SC_SLOTS)])


@functools.cache
def _compact_call():
    return pl.kernel(
        _compact_body,
        out_type=jax.ShapeDtypeStruct((_OUT_PAD,), jnp.int32),
        mesh=plsc.VectorSubcoreMesh(core_axis_name="c", subcore_axis_name="s"),
        compiler_params=pltpu.CompilerParams(needs_layout_passes=False),
        scratch_types=[
            pltpu.VMEM((_NP,), jnp.int32),
            pltpu.VMEM((_NP,), jnp.int32),
            pltpu.VMEM((_SC_SLOTS,), jnp.int32),
        ],
    )


def kernel(boxes, scores, iou_threshold):
    order = jnp.argsort(-scores)
    bs = boxes[order]
    pad = _NP - _N
    x1 = jnp.pad(bs[:, 0], (0, pad))
    y1 = jnp.pad(bs[:, 1], (0, pad))
    x2 = jnp.pad(bs[:, 2], (0, pad))
    y2 = jnp.pad(bs[:, 3], (0, pad))
    area = (x2 - x1) * (y2 - y1)
    rowdata = jnp.stack([x1, y1, x2, y2, area])
    coldata = rowdata.T
    thr = jnp.reshape(iou_threshold.astype(jnp.float32), (1,))

    c = _run_suppression(coldata, rowdata, thr).reshape(_NP)
    orig = jnp.pad(order.astype(jnp.int32), (0, pad))
    out = _compact_call()(c, orig)
    return out[:_MAX_OUT]
